# baseline (speedup 1.0000x reference)
import jax
import jax.numpy as jnp
from jax.experimental import pallas as pl
from jax.experimental.pallas import tpu as pltpu

B, N, D = 64, 32768, 64
TOP_K = 25
BLK = 2048
NB = N // BLK
CH = 128
NCH = N // CH
CAND = TOP_K * CH


def _row_normalize(x):
    n = jnp.sqrt(jnp.sum(x * x, axis=1, keepdims=True))
    n1 = jnp.maximum(n, 1e-12)
    n2 = jnp.maximum(n / n1, 1e-8)
    return x * (1.0 / (n1 * n2))


def _col_scale(x):
    n = jnp.sqrt(jnp.sum(x * x, axis=0, keepdims=True))
    n1 = jnp.maximum(n, 1e-12)
    n2 = jnp.maximum(n / n1, 1e-8)
    return 1.0 / (n1 * n2)


def _body(q1_ref, k1_ref, q2_ref, k2_ref, temp_ref, out_ref,
          work_ref, e2_ref, cm_ref, ci_ref, cand_ref):
    q1n = _row_normalize(q1_ref[...])
    q2n = _row_normalize(q2_ref[...])

    z1 = jnp.zeros((B, 1), jnp.float32)
    z2 = jnp.zeros((B, 1), jnp.float32)
    for b in range(NB):
        sl = pl.ds(b * BLK, BLK)
        k1b = k1_ref[:, sl]
        k2b = k2_ref[:, sl]
        k1n = k1b * _col_scale(k1b)
        k2n = k2b * _col_scale(k2b)
        s1 = jax.lax.dot_general(q1n, k1n, (((1,), (0,)), ((), ())),
                                 precision=jax.lax.Precision.HIGHEST,
                                 preferred_element_type=jnp.float32)
        s2 = jax.lax.dot_general(q2n, k2n, (((1,), (0,)), ((), ())),
                                 precision=jax.lax.Precision.HIGHEST,
                                 preferred_element_type=jnp.float32)
        e1 = jnp.exp(jnp.clip(s1, 0.0, 1.0))
        e2 = jnp.exp(jnp.clip(s2, 0.0, 1.0))
        work_ref[:, sl] = e1
        e2_ref[:, sl] = e2
        z1 = z1 + jnp.sum(e1, axis=1, keepdims=True)
        z2 = z2 + jnp.sum(e2, axis=1, keepdims=True)

    a = jax.nn.sigmoid(temp_ref[...])
    c1 = a / z1
    c2 = (1.0 - a) / z2

    for b in range(NB):
        sl = pl.ds(b * BLK, BLK)
        att = work_ref[:, sl] * c1 + e2_ref[:, sl] * c2
        work_ref[:, sl] = att
        for c in range(BLK // CH):
            cm_ref[:, pl.ds(b * (BLK // CH) + c, 1)] = jnp.max(
                att[:, c * CH:(c + 1) * CH], axis=1, keepdims=True)

    colc_t = jax.lax.broadcasted_iota(jnp.int32, (NCH, 1), 0).astype(jnp.float32)
    krow_t = jax.lax.broadcasted_iota(jnp.int32, (TOP_K, 1), 0)

    def _citer(k, carry):
        idxprev, acc, cm = carry
        cm = jnp.where(colc_t == idxprev, -cm, cm)
        m = jnp.max(cm, axis=0, keepdims=True)
        idx = jnp.min(jnp.where(cm == m, colc_t, 1e9), axis=0, keepdims=True)
        return idx, jnp.where(krow_t == k, idx.astype(jnp.int32), acc), cm

    _, ci_t, _ = jax.lax.fori_loop(
        0, TOP_K, _citer,
        (jnp.full((1, B), -1.0, jnp.float32),
         jnp.zeros((TOP_K, B), jnp.int32),
         cm_ref[...].T))
    ci_ref[...] = ci_t.T

    for r in range(B):
        for k in range(TOP_K):
            c = ci_ref[r, k]
            cand_ref[pl.ds(r, 1), pl.ds(k * CH, CH)] = \
                work_ref[pl.ds(r, 1), pl.ds(pl.multiple_of(c * CH, CH), CH)]

    dcol = jax.lax.broadcasted_iota(jnp.int32, (CH, 1), 0).astype(jnp.float32)
    cif = ci_t.astype(jnp.float32)
    gcol_t = jnp.concatenate(
        [cif[k:k + 1, :] * float(CH) + dcol for k in range(TOP_K)], axis=0)

    def _iter(k, carry):
        idxprev, selfidx, _, w = carry
        w = jnp.where(gcol_t == idxprev, -w, w)
        m = jnp.max(w, axis=0, keepdims=True)
        idx = jnp.min(jnp.where(w == m, gcol_t, 1e9), axis=0, keepdims=True)
        return idx, jnp.where(k == 0, idx, selfidx), m, w

    _, selfidx_t, v25_t, _ = jax.lax.fori_loop(
        0, TOP_K, _iter,
        (jnp.full((1, B), -1.0, jnp.float32),
         jnp.full((1, B), -1.0, jnp.float32),
         jnp.zeros((1, B), jnp.float32),
         cand_ref[...].T))
    selfidx = selfidx_t.T
    v25 = v25_t.T

    for b in range(NB):
        sl = pl.ds(b * BLK, BLK)
        col = (b * BLK + jax.lax.broadcasted_iota(jnp.int32, (B, BLK), 1)
               ).astype(jnp.float32)
        att = work_ref[:, sl]
        picked = (att >= v25).astype(jnp.float32)
        colsel = jnp.max(picked, axis=0, keepdims=True)
        selfm = jnp.max((col == selfidx).astype(jnp.float32),
                        axis=0, keepdims=True)
        out_ref[:, sl] = att * (colsel * (1.0 - selfm))


def kernel(q1, k1, q2, k2, temp):
    return pl.pallas_call(
        _body,
        out_shape=jax.ShapeDtypeStruct((B, N), jnp.float32),
        scratch_shapes=[
            pltpu.VMEM((B, N), jnp.float32),
            pltpu.VMEM((B, N), jnp.float32),
            pltpu.VMEM((B, NCH), jnp.float32),
            pltpu.VMEM((B, TOP_K), jnp.int32),
            pltpu.VMEM((B, CAND), jnp.float32),
        ],
        compiler_params=pltpu.CompilerParams(
            vmem_limit_bytes=100 * 1024 * 1024,
        ),
    )(q1, k1.T, q2, k2.T, temp.reshape(1, 1))


# speedup vs baseline: 1.2798x; 1.2798x over previous
import jax
import jax.numpy as jnp
from jax.experimental import pallas as pl
from jax.experimental.pallas import tpu as pltpu

B, N, D = 64, 32768, 64
TOP_K = 25
BLK = 2048
NB = N // BLK
CH = 128
NCH = N // CH
CAND = TOP_K * CH


def _row_normalize(x):
    n = jnp.sqrt(jnp.sum(x * x, axis=1, keepdims=True))
    n1 = jnp.maximum(n, 1e-12)
    n2 = jnp.maximum(n / n1, 1e-8)
    return x * (1.0 / (n1 * n2))


def _col_scale(x):
    n = jnp.sqrt(jnp.sum(x * x, axis=0, keepdims=True))
    n1 = jnp.maximum(n, 1e-12)
    n2 = jnp.maximum(n / n1, 1e-8)
    return 1.0 / (n1 * n2)


def _body(q1_ref, k1_ref, q2_ref, k2_ref, temp_ref, out_ref,
          work_ref, e2_ref, cm_ref, ci_ref, cand_ref, gcol_ref):
    q1n = _row_normalize(q1_ref[...])
    q2n = _row_normalize(q2_ref[...])

    z1 = jnp.zeros((B, 1), jnp.float32)
    z2 = jnp.zeros((B, 1), jnp.float32)
    for b in range(NB):
        sl = pl.ds(b * BLK, BLK)
        k1b = k1_ref[:, sl]
        k2b = k2_ref[:, sl]
        k1n = k1b * _col_scale(k1b)
        k2n = k2b * _col_scale(k2b)
        s1 = jax.lax.dot_general(q1n, k1n, (((1,), (0,)), ((), ())),
                                 precision=jax.lax.Precision.HIGHEST,
                                 preferred_element_type=jnp.float32)
        s2 = jax.lax.dot_general(q2n, k2n, (((1,), (0,)), ((), ())),
                                 precision=jax.lax.Precision.HIGHEST,
                                 preferred_element_type=jnp.float32)
        e1 = jnp.exp(jnp.clip(s1, 0.0, 1.0))
        e2 = jnp.exp(jnp.clip(s2, 0.0, 1.0))
        work_ref[:, sl] = e1
        e2_ref[:, sl] = e2
        z1 = z1 + jnp.sum(e1, axis=1, keepdims=True)
        z2 = z2 + jnp.sum(e2, axis=1, keepdims=True)

    a = jax.nn.sigmoid(temp_ref[...])
    c1 = a / z1
    c2 = (1.0 - a) / z2

    for b in range(NB):
        sl = pl.ds(b * BLK, BLK)
        att = work_ref[:, sl] * c1 + e2_ref[:, sl] * c2
        work_ref[:, sl] = att
        for c in range(BLK // CH):
            cm_ref[:, pl.ds(b * (BLK // CH) + c, 1)] = jnp.max(
                att[:, c * CH:(c + 1) * CH], axis=1, keepdims=True)

    colc = jax.lax.broadcasted_iota(jnp.int32, (B, NCH), 1).astype(jnp.float32)

    kcol = jax.lax.broadcasted_iota(jnp.int32, (B, TOP_K), 1)

    def _citer(k, carry):
        idxprev, acc = carry
        cm = jnp.where(colc == idxprev, -cm_ref[...], cm_ref[...])
        cm_ref[...] = cm
        m = jnp.max(cm, axis=1, keepdims=True)
        idx = jnp.min(jnp.where(cm == m, colc, 1e9), axis=1, keepdims=True)
        return idx, jnp.where(kcol == k, idx.astype(jnp.int32), acc)

    _, ci_all = jax.lax.fori_loop(
        0, TOP_K, _citer,
        (jnp.full((B, 1), -1.0, jnp.float32),
         jnp.zeros((B, TOP_K), jnp.int32)))
    ci_ref[...] = ci_all

    lcol = jax.lax.broadcasted_iota(jnp.int32, (1, CH), 1).astype(jnp.float32)
    for r in range(B):
        for k in range(TOP_K):
            c = ci_ref[r, k]
            cand_ref[pl.ds(r, 1), pl.ds(k * CH, CH)] = \
                work_ref[pl.ds(r, 1), pl.ds(pl.multiple_of(c * CH, CH), CH)]
            gcol_ref[pl.ds(r, 1), pl.ds(k * CH, CH)] = \
                lcol + (c * CH).astype(jnp.float32)

    gcol = gcol_ref[...]

    def _iter(k, carry):
        idxprev, selfidx, _ = carry
        w = jnp.where(gcol == idxprev, -cand_ref[...], cand_ref[...])
        cand_ref[...] = w
        m = jnp.max(w, axis=1, keepdims=True)
        idx = jnp.min(jnp.where(w == m, gcol, 1e9), axis=1, keepdims=True)
        return idx, jnp.where(k == 0, idx, selfidx), m

    _, selfidx, v25 = jax.lax.fori_loop(
        0, TOP_K, _iter,
        (jnp.full((B, 1), -1.0, jnp.float32),
         jnp.full((B, 1), -1.0, jnp.float32),
         jnp.zeros((B, 1), jnp.float32)))

    for b in range(NB):
        sl = pl.ds(b * BLK, BLK)
        col = (b * BLK + jax.lax.broadcasted_iota(jnp.int32, (B, BLK), 1)
               ).astype(jnp.float32)
        att = work_ref[:, sl]
        picked = (att >= v25).astype(jnp.float32)
        colsel = jnp.max(picked, axis=0, keepdims=True)
        selfm = jnp.max((col == selfidx).astype(jnp.float32),
                        axis=0, keepdims=True)
        out_ref[:, sl] = att * (colsel * (1.0 - selfm))


def kernel(q1, k1, q2, k2, temp):
    return pl.pallas_call(
        _body,
        out_shape=jax.ShapeDtypeStruct((B, N), jnp.float32),
        scratch_shapes=[
            pltpu.VMEM((B, N), jnp.float32),
            pltpu.VMEM((B, N), jnp.float32),
            pltpu.VMEM((B, NCH), jnp.float32),
            pltpu.VMEM((B, TOP_K), jnp.int32),
            pltpu.VMEM((B, CAND), jnp.float32),
            pltpu.VMEM((B, CAND), jnp.float32),
        ],
        compiler_params=pltpu.CompilerParams(
            vmem_limit_bytes=100 * 1024 * 1024,
        ),
    )(q1, k1.T, q2, k2.T, temp.reshape(1, 1))
